# baseline (speedup 1.0000x reference)
import jax
import jax.numpy as jnp
from jax.experimental import pallas as pl
from jax.experimental.pallas import tpu as pltpu


def _pool_kernel(h_ref, w_ref, o_ref):
    t = jnp.tanh(h_ref[...])
    prod = t * w_ref[...].reshape(1, 1, -1)
    scores = jnp.sum(prod, axis=-1)
    m = jnp.max(scores, axis=-1, keepdims=True)
    e = jnp.exp(scores - m)
    den = jnp.sum(e, axis=-1, keepdims=True)
    o_ref[...] = e / den


def kernel(H, weight, bias):
    B, S, D = H.shape
    del bias
    w32 = weight.reshape(1, D).astype(jnp.float32)

    TB = min(B, 64)
    while B % TB:
        TB //= 2
    grid = (pl.cdiv(B, TB),)
    out = pl.pallas_call(
        _pool_kernel,
        out_shape=jax.ShapeDtypeStruct((B, S), jnp.float32),
        grid=grid,
        in_specs=[
            pl.BlockSpec((TB, S, D), lambda b: (b, 0, 0)),
            pl.BlockSpec((1, D), lambda b: (0, 0)),
        ],
        out_specs=pl.BlockSpec((TB, S), lambda b: (b, 0)),
        compiler_params=pltpu.CompilerParams(
            dimension_semantics=("parallel",),
            vmem_limit_bytes=64 << 20,
        ),
    )(H, w32)
    return out.reshape(B, 1, S).astype(H.dtype)


# speedup vs baseline: 1.4008x; 1.4008x over previous
import jax
import jax.numpy as jnp
from jax.experimental import pallas as pl
from jax.experimental.pallas import tpu as pltpu


def _pool_kernel(h_ref, w_ref, o_ref, s_ref):
    t = jnp.tanh(h_ref[...])
    prod = t * w_ref[...].reshape(1, 1, -1)
    s_ref[...] = jnp.sum(prod, axis=-1)
    scores = s_ref[...]
    m = jnp.max(scores, axis=-1, keepdims=True)
    e = jnp.exp(scores - m)
    den = jnp.sum(e, axis=-1, keepdims=True)
    o_ref[...] = (e / den)[:, None, :]


def kernel(H, weight, bias):
    B, S, D = H.shape
    del bias
    w32 = weight.reshape(1, D).astype(jnp.float32)

    TB = min(B, 64)
    while B % TB:
        TB //= 2
    grid = (pl.cdiv(B, TB),)
    out = pl.pallas_call(
        _pool_kernel,
        out_shape=jax.ShapeDtypeStruct((B, 1, S), H.dtype),
        grid=grid,
        in_specs=[
            pl.BlockSpec((TB, S, D), lambda b: (b, 0, 0)),
            pl.BlockSpec((1, D), lambda b: (0, 0)),
        ],
        out_specs=pl.BlockSpec((TB, 1, S), lambda b: (b, 0, 0)),
        scratch_shapes=[pltpu.VMEM((TB, S), jnp.float32)],
        compiler_params=pltpu.CompilerParams(
            dimension_semantics=("parallel",),
            vmem_limit_bytes=64 << 20,
        ),
    )(H, w32)
    return out


# speedup vs baseline: 7.5543x; 5.3930x over previous
import jax
import jax.numpy as jnp
from jax.experimental import pallas as pl
from jax.experimental.pallas import tpu as pltpu


def _pool_kernel(ht_ref, w_ref, o_ref):
    t = jnp.tanh(ht_ref[...])
    prod = t * w_ref[...]
    scores = jnp.sum(prod, axis=1)
    m = jnp.max(scores, axis=-1, keepdims=True)
    e = jnp.exp(scores - m)
    den = jnp.sum(e, axis=-1, keepdims=True)
    o_ref[...] = (e / den)[:, None, :]


def kernel(H, weight, bias):
    B, S, D = H.shape
    del bias
    Ht = jnp.transpose(H, (0, 2, 1))
    w_bc = jnp.broadcast_to(
        weight.reshape(D, 1).astype(jnp.float32), (D, S)
    ).reshape(1, D, S)

    TB = min(B, 128)
    while B % TB:
        TB //= 2
    grid = (pl.cdiv(B, TB),)
    out = pl.pallas_call(
        _pool_kernel,
        out_shape=jax.ShapeDtypeStruct((B, 1, S), H.dtype),
        grid=grid,
        in_specs=[
            pl.BlockSpec((TB, D, S), lambda b: (b, 0, 0)),
            pl.BlockSpec((1, D, S), lambda b: (0, 0, 0)),
        ],
        out_specs=pl.BlockSpec((TB, 1, S), lambda b: (b, 0, 0)),
        compiler_params=pltpu.CompilerParams(
            dimension_semantics=("parallel",),
            vmem_limit_bytes=64 << 20,
        ),
    )(Ht, w_bc)
    return out


# speedup vs baseline: 8.4571x; 1.1195x over previous
import jax
import jax.numpy as jnp
from jax.experimental import pallas as pl
from jax.experimental.pallas import tpu as pltpu


def _pool_kernel(ht_ref, w_ref, o_ref):
    t = jnp.tanh(ht_ref[...])
    prod = t * w_ref[...]
    scores = jnp.sum(prod, axis=1)
    m = jnp.max(scores, axis=-1, keepdims=True)
    e = jnp.exp(scores - m)
    den = jnp.sum(e, axis=-1, keepdims=True)
    o_ref[...] = (e / den)[:, None, :]


def kernel(H, weight, bias):
    B, S, D = H.shape
    del bias
    Ht = jnp.transpose(H, (0, 2, 1))
    w_bc = jnp.broadcast_to(
        weight.reshape(D, 1).astype(jnp.float32), (D, S)
    ).reshape(1, D, S)

    TB = min(B, 256)
    while B % TB:
        TB //= 2
    grid = (pl.cdiv(B, TB),)
    out = pl.pallas_call(
        _pool_kernel,
        out_shape=jax.ShapeDtypeStruct((B, 1, S), H.dtype),
        grid=grid,
        in_specs=[
            pl.BlockSpec((TB, D, S), lambda b: (b, 0, 0)),
            pl.BlockSpec((1, D, S), lambda b: (0, 0, 0)),
        ],
        out_specs=pl.BlockSpec((TB, 1, S), lambda b: (b, 0, 0)),
        compiler_params=pltpu.CompilerParams(
            dimension_semantics=("parallel",),
            vmem_limit_bytes=64 << 20,
        ),
    )(Ht, w_bc)
    return out
